# baseline (speedup 1.0000x reference)
import functools

import jax
import jax.numpy as jnp
from jax import lax
from jax.experimental import pallas as pl
from jax.experimental.pallas import tpu as pltpu
from jax.experimental.pallas import tpu_sc as plsc

_NC = 1
_NS = 16
_TILE_ROWS = 16


def kernel(x, pos_table):
    batch = x.shape[0]
    seq_len = x.shape[1]
    emb = pos_table.shape[1]
    flat = seq_len * emb
    tab = pos_table[:seq_len].reshape(flat)

    nw = _NC * _NS
    rows_per_w = batch // nw
    ncopies = rows_per_w // _TILE_ROWS

    mesh = plsc.VectorSubcoreMesh(
        core_axis_name="c", subcore_axis_name="s", num_cores=_NC
    )

    @functools.partial(
        pl.kernel,
        out_type=jax.ShapeDtypeStruct((batch, flat), pos_table.dtype),
        mesh=mesh,
        scratch_types=[
            pltpu.VMEM((_TILE_ROWS, flat), pos_table.dtype),
            pltpu.SemaphoreType.DMA,
        ],
    )
    def sc_broadcast(tab_hbm, out_hbm, tile_v, sem):
        wid = lax.axis_index("s") * _NC + lax.axis_index("c")
        base = wid * rows_per_w
        pltpu.async_copy(tab_hbm, tile_v.at[0], sem).wait()

        def _rep(i, _):
            chunk = tile_v[0, pl.ds(i * 16, 16)]
            for r in range(1, _TILE_ROWS):
                tile_v[r, pl.ds(i * 16, 16)] = chunk
            return _

        lax.fori_loop(0, flat // 16, _rep, None)
        for j in range(ncopies):
            pltpu.async_copy(
                tile_v,
                out_hbm.at[pl.ds(base + j * _TILE_ROWS, _TILE_ROWS), :],
                sem,
            )
        for j in range(ncopies):
            pltpu.make_async_copy(
                tile_v,
                out_hbm.at[pl.ds(base + j * _TILE_ROWS, _TILE_ROWS), :],
                sem,
            ).wait()

    out = sc_broadcast(tab)
    return out.reshape(batch, seq_len, emb)


# speedup vs baseline: 1.1688x; 1.1688x over previous
import functools

import jax
import jax.numpy as jnp
from jax import lax
from jax.experimental import pallas as pl
from jax.experimental.pallas import tpu as pltpu
from jax.experimental.pallas import tpu_sc as plsc

_NC = 2
_NS = 16
_TILE_ROWS = 16


def kernel(x, pos_table):
    batch = x.shape[0]
    seq_len = x.shape[1]
    emb = pos_table.shape[1]
    flat = seq_len * emb
    tab = pos_table[:seq_len].reshape(flat)

    nw = _NC * _NS
    rows_per_w = batch // nw
    ncopies = rows_per_w // _TILE_ROWS

    mesh = plsc.VectorSubcoreMesh(
        core_axis_name="c", subcore_axis_name="s", num_cores=_NC
    )

    @functools.partial(
        pl.kernel,
        out_type=jax.ShapeDtypeStruct((batch, flat), pos_table.dtype),
        mesh=mesh,
        scratch_types=[
            pltpu.VMEM((_TILE_ROWS, flat), pos_table.dtype),
            pltpu.SemaphoreType.DMA,
        ],
        compiler_params=pltpu.CompilerParams(
            disable_bounds_checks=True,
            disable_semaphore_checks=True,
            skip_device_barrier=True,
        ),
    )
    def sc_broadcast(tab_hbm, out_hbm, tile_v, sem):
        wid = lax.axis_index("s") * _NC + lax.axis_index("c")
        base = wid * rows_per_w
        pltpu.async_copy(tab_hbm, tile_v.at[0], sem).wait()

        def _rep(i, _):
            chunk = tile_v[0, pl.ds(i * 16, 16)]
            for r in range(1, _TILE_ROWS):
                tile_v[r, pl.ds(i * 16, 16)] = chunk
            return _

        lax.fori_loop(0, flat // 16, _rep, None)
        for j in range(ncopies):
            pltpu.async_copy(
                tile_v,
                out_hbm.at[pl.ds(base + j * _TILE_ROWS, _TILE_ROWS), :],
                sem,
            )
        for j in range(ncopies):
            pltpu.make_async_copy(
                tile_v,
                out_hbm.at[pl.ds(base + j * _TILE_ROWS, _TILE_ROWS), :],
                sem,
            ).wait()

    out = sc_broadcast(tab)
    return out.reshape(batch, seq_len, emb)
